# Initial kernel scaffold; baseline (speedup 1.0000x reference)
import functools

import jax
import jax.numpy as jnp
from jax import lax
from jax.experimental import pallas as pl
from jax.experimental.pallas import tpu as pltpu
from jax.experimental.pallas import tpu_sc as plsc

D0, D1, D2 = 160, 160, 160
C = 12
CPAD = 16
NROWS = D0 * D1 * D2
NWORKERS = 32
CHUNK = 256
NIDX = 8 * CHUNK
GATHER_SLICE = 128


def _build_sc_kernel(n_pts):
    ppw = n_pts // NWORKERS
    nchunks = ppw // CHUNK
    mesh = plsc.VectorSubcoreMesh(core_axis_name="c", subcore_axis_name="s")

    @functools.partial(
        pl.kernel,
        mesh=mesh,
        out_type=jax.ShapeDtypeStruct((n_pts * C,), jnp.float32),
        scratch_types=[
            pltpu.VMEM((8, 16), jnp.float32),
            pltpu.VMEM((3 * CHUNK,), jnp.float32),
            pltpu.VMEM((NIDX,), jnp.int32),
            pltpu.VMEM((NIDX,), jnp.float32),
            pltpu.VMEM((NIDX, CPAD), jnp.float32),
            pltpu.VMEM((CHUNK * C,), jnp.float32),
            pltpu.SemaphoreType.DMA,
        ],
    )
    def k(q_hbm, table_hbm, consts_hbm, out_hbm,
          consts_v, q_v, idx_v, w_v, rows_v, out_v, sem):
        cid = lax.axis_index("c")
        sid = lax.axis_index("s")
        wid = sid * 2 + cid
        base0 = wid * ppw

        pltpu.sync_copy(consts_hbm, consts_v)
        lane = lax.iota(jnp.int32, 16)
        lane3 = lane * 3
        laneC = lane * C
        sx = consts_v[0, :]
        sy = consts_v[1, :]
        sz = consts_v[2, :]
        ox = consts_v[3, :]
        oy = consts_v[4, :]
        oz = consts_v[5, :]

        def chunk_body(ci, _):
            base = base0 + ci * CHUNK
            pltpu.sync_copy(q_hbm.at[pl.ds(base * 3, 3 * CHUNK)], q_v)

            def grp_a(g, _):
                off3 = g * 48
                x = plsc.load_gather(q_v, [lane3 + off3])
                y = plsc.load_gather(q_v, [lane3 + (off3 + 1)])
                z = plsc.load_gather(q_v, [lane3 + (off3 + 2)])
                qx = x * sx + ox
                qy = y * sy + oy
                qz = z * sz + oz
                ix = jnp.minimum(jnp.maximum(qx.astype(jnp.int32), 0), D0 - 2)
                iy = jnp.minimum(jnp.maximum(qy.astype(jnp.int32), 0), D1 - 2)
                iz = jnp.minimum(jnp.maximum(qz.astype(jnp.int32), 0), D2 - 2)
                fx = qx - ix.astype(jnp.float32)
                fy = qy - iy.astype(jnp.float32)
                fz = qz - iz.astype(jnp.float32)
                flat = ix * (D1 * D2) + iy * D2 + iz
                gx0 = 1.0 - fx
                gy0 = 1.0 - fy
                gz0 = 1.0 - fz
                wxy = (gx0 * gy0, gx0 * fy, fx * gy0, fx * fy)
                p16 = g * 16
                for kk in range(8):
                    dx, dy, dz = kk >> 2, (kk >> 1) & 1, kk & 1
                    corner_off = dx * (D1 * D2) + dy * D2 + dz
                    idx_v[pl.ds(kk * CHUNK + p16, 16)] = flat + corner_off
                    wz = fz if dz else gz0
                    w_v[pl.ds(kk * CHUNK + p16, 16)] = wxy[2 * dx + dy] * wz
                return 0

            lax.fori_loop(0, CHUNK // 16, grp_a, 0)

            copies = []
            for j in range(NIDX // GATHER_SLICE):
                copies.append(pltpu.async_copy(
                    table_hbm.at[idx_v.at[pl.ds(j * GATHER_SLICE, GATHER_SLICE)]],
                    rows_v.at[pl.ds(j * GATHER_SLICE, GATHER_SLICE), :],
                    sem))
            for cp in copies:
                cp.wait()

            def grp_c(g, _):
                p16 = g * 16
                ws = [w_v[pl.ds(kk * CHUNK + p16, 16)] for kk in range(8)]
                row0 = lane + p16
                for ch in range(C):
                    col = jnp.full((16,), ch, jnp.int32)
                    acc = ws[0] * plsc.load_gather(rows_v, [row0, col])
                    for kk in range(1, 8):
                        v = plsc.load_gather(rows_v, [row0 + kk * CHUNK, col])
                        acc = acc + ws[kk] * v
                    plsc.store_scatter(out_v, [laneC + (p16 * C + ch)], acc)
                return 0

            lax.fori_loop(0, CHUNK // 16, grp_c, 0)
            pltpu.sync_copy(out_v, out_hbm.at[pl.ds(base * C, CHUNK * C)])
            return 0

        lax.fori_loop(0, nchunks, chunk_body, 0)

    return k


def kernel(xyz, grid, xyz_min, xyz_max):
    shape = xyz.shape[:-1]
    pts = xyz.reshape(-1, 3)
    n_pts = pts.shape[0]

    table = jnp.pad(jnp.transpose(grid, (1, 2, 3, 0)),
                    ((0, 0), (0, 0), (0, 0), (0, CPAD - grid.shape[0])))
    table = table.reshape(NROWS, CPAD)

    sizes = jnp.array([D0 - 1, D1 - 1, D2 - 1], dtype=jnp.float32)
    scale = sizes / (xyz_max - xyz_min)
    off = -xyz_min * scale
    consts = jnp.zeros((8, 16), jnp.float32)
    consts = consts.at[0:3, :].set(jnp.broadcast_to(scale[:, None], (3, 16)))
    consts = consts.at[3:6, :].set(jnp.broadcast_to(off[:, None], (3, 16)))

    q_flat = pts.reshape(-1)
    out = _build_sc_kernel(n_pts)(q_flat, table, consts)
    out = out.reshape(*shape, C)
    return out


# speedup vs baseline: 1.0159x; 1.0159x over previous
import functools

import jax
import jax.numpy as jnp
from jax import lax
from jax.experimental import pallas as pl
from jax.experimental.pallas import tpu as pltpu
from jax.experimental.pallas import tpu_sc as plsc

D0, D1, D2 = 160, 160, 160
C = 12
CPAD = 16
NROWS = D0 * D1 * D2
NWORKERS = 32
CHUNK = 256
NIDX = 8 * CHUNK
GATHER_SLICE = 128


def _build_sc_kernel(n_pts):
    ppw = n_pts // NWORKERS
    nchunks = ppw // CHUNK
    mesh = plsc.VectorSubcoreMesh(core_axis_name="c", subcore_axis_name="s")

    @functools.partial(
        pl.kernel,
        mesh=mesh,
        compiler_params=pltpu.CompilerParams(
            needs_layout_passes=False, use_tc_tiling_on_sc=False),
        out_type=jax.ShapeDtypeStruct((n_pts * C,), jnp.float32),
        scratch_types=[
            pltpu.VMEM((8, 16), jnp.float32),
            pltpu.VMEM((3 * CHUNK,), jnp.float32),
            pltpu.VMEM((NIDX,), jnp.int32),
            pltpu.VMEM((NIDX,), jnp.float32),
            pltpu.VMEM((NIDX, CPAD), jnp.float32),
            pltpu.VMEM((CHUNK * C,), jnp.float32),
            pltpu.SemaphoreType.DMA,
        ],
    )
    def k(q_hbm, table_hbm, consts_hbm, out_hbm,
          consts_v, q_v, idx_v, w_v, rows_v, out_v, sem):
        cid = lax.axis_index("c")
        sid = lax.axis_index("s")
        wid = sid * 2 + cid
        base0 = wid * ppw

        pltpu.sync_copy(consts_hbm, consts_v)
        lane = lax.iota(jnp.int32, 16)
        lane3 = lane * 3
        laneC = lane * C
        sx = consts_v[0, :]
        sy = consts_v[1, :]
        sz = consts_v[2, :]
        ox = consts_v[3, :]
        oy = consts_v[4, :]
        oz = consts_v[5, :]

        def chunk_body(ci, _):
            base = base0 + ci * CHUNK
            pltpu.sync_copy(q_hbm.at[pl.ds(base * 3, 3 * CHUNK)], q_v)

            def grp_a(g, _):
                off3 = g * 48
                x = plsc.load_gather(q_v, [lane3 + off3])
                y = plsc.load_gather(q_v, [lane3 + (off3 + 1)])
                z = plsc.load_gather(q_v, [lane3 + (off3 + 2)])
                qx = x * sx + ox
                qy = y * sy + oy
                qz = z * sz + oz
                ix = jnp.minimum(jnp.maximum(qx.astype(jnp.int32), 0), D0 - 2)
                iy = jnp.minimum(jnp.maximum(qy.astype(jnp.int32), 0), D1 - 2)
                iz = jnp.minimum(jnp.maximum(qz.astype(jnp.int32), 0), D2 - 2)
                fx = qx - ix.astype(jnp.float32)
                fy = qy - iy.astype(jnp.float32)
                fz = qz - iz.astype(jnp.float32)
                flat = ix * (D1 * D2) + iy * D2 + iz
                gx0 = 1.0 - fx
                gy0 = 1.0 - fy
                gz0 = 1.0 - fz
                wxy = (gx0 * gy0, gx0 * fy, fx * gy0, fx * fy)
                p16 = g * 16
                for kk in range(8):
                    dx, dy, dz = kk >> 2, (kk >> 1) & 1, kk & 1
                    corner_off = dx * (D1 * D2) + dy * D2 + dz
                    idx_v[pl.ds(kk * CHUNK + p16, 16)] = flat + corner_off
                    wz = fz if dz else gz0
                    w_v[pl.ds(kk * CHUNK + p16, 16)] = wxy[2 * dx + dy] * wz
                return 0

            lax.fori_loop(0, CHUNK // 16, grp_a, 0)

            copies = []
            for j in range(NIDX // GATHER_SLICE):
                copies.append(pltpu.async_copy(
                    table_hbm.at[idx_v.at[pl.ds(j * GATHER_SLICE, GATHER_SLICE)]],
                    rows_v.at[pl.ds(j * GATHER_SLICE, GATHER_SLICE), :],
                    sem))
            for cp in copies:
                cp.wait()

            def grp_c(g, _):
                p16 = g * 16
                ws = [w_v[pl.ds(kk * CHUNK + p16, 16)] for kk in range(8)]
                row0 = lane + p16
                for ch in range(C):
                    col = jnp.full((16,), ch, jnp.int32)
                    acc = ws[0] * plsc.load_gather(rows_v, [row0, col])
                    for kk in range(1, 8):
                        v = plsc.load_gather(rows_v, [row0 + kk * CHUNK, col])
                        acc = acc + ws[kk] * v
                    plsc.store_scatter(out_v, [laneC + (p16 * C + ch)], acc)
                return 0

            lax.fori_loop(0, CHUNK // 16, grp_c, 0)
            pltpu.sync_copy(out_v, out_hbm.at[pl.ds(base * C, CHUNK * C)])
            return 0

        lax.fori_loop(0, nchunks, chunk_body, 0)

    return k


def kernel(xyz, grid, xyz_min, xyz_max):
    shape = xyz.shape[:-1]
    pts = xyz.reshape(-1, 3)
    n_pts = pts.shape[0]

    table = jnp.pad(jnp.transpose(grid, (1, 2, 3, 0)),
                    ((0, 0), (0, 0), (0, 0), (0, CPAD - grid.shape[0])))
    table = table.reshape(NROWS, CPAD)

    sizes = jnp.array([D0 - 1, D1 - 1, D2 - 1], dtype=jnp.float32)
    scale = sizes / (xyz_max - xyz_min)
    off = -xyz_min * scale
    consts = jnp.zeros((8, 16), jnp.float32)
    consts = consts.at[0:3, :].set(jnp.broadcast_to(scale[:, None], (3, 16)))
    consts = consts.at[3:6, :].set(jnp.broadcast_to(off[:, None], (3, 16)))

    q_flat = pts.reshape(-1)
    out = _build_sc_kernel(n_pts)(q_flat, table, consts)
    out = out.reshape(*shape, C)
    return out
